# Initial kernel scaffold; baseline (speedup 1.0000x reference)
import jax
import jax.numpy as jnp
from jax.experimental import pallas as pl


def kernel(x, W, b, centroids):
    raise NotImplementedError("write your pallas kernel here")


# speedup vs baseline: 2.2157x; 2.2157x over previous
import jax
import jax.numpy as jnp
from jax.experimental import pallas as pl

_D = 2048
_VEC = 8
_K = 256
_NTOK = 64
_NV = (_D * _D) // _VEC
_VB = 8192
_CB = 128


def _vq_kernel(wv_ref, c_ref, c2_ref, q_ref):
    v = wv_ref[...]
    c = c_ref[...]
    c2 = c2_ref[...]

    norms = jnp.sqrt(jnp.sum(v * v, axis=1, keepdims=True)) + 1e-8
    nrm = v / norms
    a2 = jnp.sum(nrm * nrm, axis=1, keepdims=True)
    dots = jax.lax.dot_general(
        nrm, c, (((1,), (1,)), ((), ())), preferred_element_type=jnp.float32
    )
    d2 = jnp.maximum(a2 + c2 - 2.0 * dots, 0.0)
    a = jnp.argmin(d2, axis=1)[:, None]
    k_iota = jax.lax.broadcasted_iota(jnp.int32, (_VB, _K), 1)
    onehot = (k_iota == a).astype(jnp.float32)
    assigned = jax.lax.dot_general(
        onehot, c, (((1,), (0,)), ((), ())),
        preferred_element_type=jnp.float32,
        precision=jax.lax.Precision.HIGHEST,
    )
    num = jnp.sum(v * assigned, axis=1, keepdims=True)
    den = jnp.sum(assigned * assigned, axis=1, keepdims=True) + 1e-8
    q_ref[...] = assigned * (num / den)


def _linear_kernel(x_ref, wq_ref, b_ref, out_ref):
    out_ref[...] = (
        jax.lax.dot_general(
            x_ref[...], wq_ref[...], (((1,), (1,)), ((), ())),
            preferred_element_type=jnp.float32,
        )
        + b_ref[...]
    )


def kernel(x, W, b, centroids):
    wv = W.reshape(-1, _VEC)
    c2 = jnp.sum(centroids * centroids, axis=1).reshape(1, _K)

    q = pl.pallas_call(
        _vq_kernel,
        grid=(_NV // _VB,),
        in_specs=[
            pl.BlockSpec((_VB, _VEC), lambda i: (i, 0)),
            pl.BlockSpec((_K, _VEC), lambda i: (0, 0)),
            pl.BlockSpec((1, _K), lambda i: (0, 0)),
        ],
        out_specs=pl.BlockSpec((_VB, _VEC), lambda i: (i, 0)),
        out_shape=jax.ShapeDtypeStruct((_NV, _VEC), jnp.float32),
    )(wv, centroids, c2)

    wq = q.reshape(_D, _D)

    return pl.pallas_call(
        _linear_kernel,
        grid=(_D // _CB,),
        in_specs=[
            pl.BlockSpec((_NTOK, _D), lambda i: (0, 0)),
            pl.BlockSpec((_CB, _D), lambda i: (i, 0)),
            pl.BlockSpec((1, _CB), lambda i: (0, i)),
        ],
        out_specs=pl.BlockSpec((_NTOK, _CB), lambda i: (0, i)),
        out_shape=jax.ShapeDtypeStruct((_NTOK, _D), jnp.float32),
    )(x, wq, b.reshape(1, _D))


# speedup vs baseline: 4.2253x; 1.9070x over previous
import jax
import jax.numpy as jnp
from jax.experimental import pallas as pl

_D = 2048
_VEC = 8
_K = 256
_NTOK = 64
_NV = (_D * _D) // _VEC
_VB = 8192
_CB = 128


def _vq_kernel(wvt_ref, c_ref, c2_ref, ccat_ref, qt_ref):
    vt = wvt_ref[...]
    c = c_ref[...]
    c2 = c2_ref[...]

    norms = jnp.sqrt(jnp.sum(vt * vt, axis=0, keepdims=True)) + 1e-8
    nrmt = vt / norms
    a2 = jnp.sum(nrmt * nrmt, axis=0, keepdims=True)
    dots = jax.lax.dot_general(
        c, nrmt, (((1,), (0,)), ((), ())), preferred_element_type=jnp.float32
    )
    d2 = jnp.maximum(a2 + c2 - 2.0 * dots, 0.0)
    a = jnp.argmin(d2, axis=0)[None, :]
    k_iota = jax.lax.broadcasted_iota(jnp.int32, (_K, _VB), 0)
    onehot = (k_iota == a).astype(jnp.bfloat16)
    cat = jax.lax.dot_general(
        ccat_ref[...], onehot, (((1,), (0,)), ((), ())),
        preferred_element_type=jnp.float32,
    )
    assigned = (cat[0:_VEC] + cat[_VEC:2 * _VEC] * (1.0 / 512.0)) \
        + cat[2 * _VEC:] * (1.0 / 131072.0)
    num = jnp.sum(vt * assigned, axis=0, keepdims=True)
    den = jnp.sum(assigned * assigned, axis=0, keepdims=True) + 1e-8
    qt_ref[...] = assigned * (num / den)


def _linear_kernel(x_ref, wq_ref, b_ref, out_ref):
    out_ref[...] = (
        jax.lax.dot_general(
            x_ref[...], wq_ref[...], (((1,), (1,)), ((), ())),
            preferred_element_type=jnp.float32,
        )
        + b_ref[...]
    )


def kernel(x, W, b, centroids):
    wvt = W.reshape(-1, _VEC).T
    c2 = jnp.sum(centroids * centroids, axis=1).reshape(_K, 1)
    mask = jnp.uint32(0xFFFF0000)
    hi_f = jax.lax.bitcast_convert_type(
        jax.lax.bitcast_convert_type(centroids, jnp.uint32) & mask,
        jnp.float32)
    r1 = centroids - hi_f
    mid_f = jax.lax.bitcast_convert_type(
        jax.lax.bitcast_convert_type(r1, jnp.uint32) & mask, jnp.float32)
    r2 = r1 - mid_f
    c_hi = hi_f.astype(jnp.bfloat16)
    c_mid = (mid_f * 512.0).astype(jnp.bfloat16)
    c_lo = (r2 * 131072.0).astype(jnp.bfloat16)
    c_cat = jnp.concatenate([c_hi.T, c_mid.T, c_lo.T], axis=0)

    qt = pl.pallas_call(
        _vq_kernel,
        grid=(_NV // _VB,),
        in_specs=[
            pl.BlockSpec((_VEC, _VB), lambda i: (0, i)),
            pl.BlockSpec((_K, _VEC), lambda i: (0, 0)),
            pl.BlockSpec((_K, 1), lambda i: (0, 0)),
            pl.BlockSpec((3 * _VEC, _K), lambda i: (0, 0)),
        ],
        out_specs=pl.BlockSpec((_VEC, _VB), lambda i: (0, i)),
        out_shape=jax.ShapeDtypeStruct((_VEC, _NV), jnp.float32),
    )(wvt, centroids, c2, c_cat)

    wq = qt.T.reshape(_D, _D)

    return pl.pallas_call(
        _linear_kernel,
        grid=(_D // _CB,),
        in_specs=[
            pl.BlockSpec((_NTOK, _D), lambda i: (0, 0)),
            pl.BlockSpec((_CB, _D), lambda i: (i, 0)),
            pl.BlockSpec((1, _CB), lambda i: (0, i)),
        ],
        out_specs=pl.BlockSpec((_NTOK, _CB), lambda i: (0, i)),
        out_shape=jax.ShapeDtypeStruct((_NTOK, _D), jnp.float32),
    )(x, wq, b.reshape(1, _D))


# speedup vs baseline: 4.7286x; 1.1191x over previous
import jax
import jax.numpy as jnp
from jax.experimental import pallas as pl

_D = 2048
_VEC = 8
_K = 256
_NTOK = 64
_NV = (_D * _D) // _VEC
_VB = 8192
_CB = 128


def _vq_kernel(wv_ref, c_ref, c2_ref, ccat_ref, q_ref):
    vt = wv_ref[...].T
    c = c_ref[...]
    c2 = c2_ref[...]

    norms = jnp.sqrt(jnp.sum(vt * vt, axis=0, keepdims=True)) + 1e-8
    nrmt = vt / norms
    a2 = jnp.sum(nrmt * nrmt, axis=0, keepdims=True)
    dots = jax.lax.dot_general(
        c, nrmt, (((1,), (0,)), ((), ())), preferred_element_type=jnp.float32
    )
    d2 = jnp.maximum(a2 + c2 - 2.0 * dots, 0.0)
    a = jnp.argmin(d2, axis=0)[None, :]
    k_iota = jax.lax.broadcasted_iota(jnp.int32, (_K, _VB), 0)
    onehot = (k_iota == a).astype(jnp.bfloat16)
    cat = jax.lax.dot_general(
        ccat_ref[...], onehot, (((1,), (0,)), ((), ())),
        preferred_element_type=jnp.float32,
    )
    assigned = (cat[0:_VEC] + cat[_VEC:2 * _VEC] * (1.0 / 512.0)) \
        + cat[2 * _VEC:] * (1.0 / 131072.0)
    num = jnp.sum(vt * assigned, axis=0, keepdims=True)
    den = jnp.sum(assigned * assigned, axis=0, keepdims=True) + 1e-8
    q_ref[...] = (assigned * (num / den)).T


def _linear_kernel(x_ref, wq_ref, b_ref, out_ref):
    out_ref[...] = (
        jax.lax.dot_general(
            x_ref[...], wq_ref[...], (((1,), (1,)), ((), ())),
            preferred_element_type=jnp.float32,
        )
        + b_ref[...]
    )


def kernel(x, W, b, centroids):
    wv = W.reshape(-1, _VEC)
    c2 = jnp.sum(centroids * centroids, axis=1).reshape(_K, 1)
    mask = jnp.uint32(0xFFFF0000)
    hi_f = jax.lax.bitcast_convert_type(
        jax.lax.bitcast_convert_type(centroids, jnp.uint32) & mask,
        jnp.float32)
    r1 = centroids - hi_f
    mid_f = jax.lax.bitcast_convert_type(
        jax.lax.bitcast_convert_type(r1, jnp.uint32) & mask, jnp.float32)
    r2 = r1 - mid_f
    c_hi = hi_f.astype(jnp.bfloat16)
    c_mid = (mid_f * 512.0).astype(jnp.bfloat16)
    c_lo = (r2 * 131072.0).astype(jnp.bfloat16)
    c_cat = jnp.concatenate([c_hi.T, c_mid.T, c_lo.T], axis=0)

    qt = pl.pallas_call(
        _vq_kernel,
        grid=(_NV // _VB,),
        in_specs=[
            pl.BlockSpec((_VB, _VEC), lambda i: (i, 0)),
            pl.BlockSpec((_K, _VEC), lambda i: (0, 0)),
            pl.BlockSpec((_K, 1), lambda i: (0, 0)),
            pl.BlockSpec((3 * _VEC, _K), lambda i: (0, 0)),
        ],
        out_specs=pl.BlockSpec((_VB, _VEC), lambda i: (i, 0)),
        out_shape=jax.ShapeDtypeStruct((_NV, _VEC), jnp.float32),
    )(wv, centroids, c2, c_cat)

    wq = qt.reshape(_D, _D)

    return pl.pallas_call(
        _linear_kernel,
        grid=(_D // _CB,),
        in_specs=[
            pl.BlockSpec((_NTOK, _D), lambda i: (0, 0)),
            pl.BlockSpec((_CB, _D), lambda i: (i, 0)),
            pl.BlockSpec((1, _CB), lambda i: (0, i)),
        ],
        out_specs=pl.BlockSpec((_NTOK, _CB), lambda i: (0, i)),
        out_shape=jax.ShapeDtypeStruct((_NTOK, _D), jnp.float32),
    )(x, wq, b.reshape(1, _D))
